# baseline (speedup 1.0000x reference)
import functools

import jax
import jax.numpy as jnp
from jax import lax
from jax.experimental import pallas as pl
from jax.experimental.pallas import tpu as pltpu
from jax.experimental.pallas import tpu_sc as plsc

NCAM = 6
C = 64
HW = 11264
TBLK = 1408
NTB = HW // TBLK
CAM_STRIDE = (NTB + 1) * TBLK
PAD_BASE = HW
NROWS = NCAM * CAM_STRIDE
K = 25000
NWRITES = NCAM * K
S_TOTAL = 160000
NCORES = 2
NSUB = 16
NWORKERS = NCORES * NSUB
S_PER_W = S_TOTAL // NWORKERS
SRC_PAD = 5008
WIN = 2000
NWIN = NWRITES // WIN
VGRP = 25
CH = 40
GCH = 5
GROUP = CH * GCH
NGRP = S_PER_W // GROUP


HB = 8


def _t1_body(f_ref, o_ref):
    b = pl.program_id(1)
    x4 = f_ref[0]
    eye = (lax.broadcasted_iota(jnp.int32, (C, C), 0)
           == lax.broadcasted_iota(jnp.int32, (C, C), 1)).astype(jnp.float32)
    o_ref[:, 64:128] = jnp.zeros((TBLK, 64), jnp.float32)
    for i in range(HB):
        xi = x4[:, i, :]
        yi = lax.dot_general(xi, eye, (((0,), (0,)), ((), ())),
                             preferred_element_type=jnp.float32,
                             precision=lax.Precision.HIGHEST)
        o_ref[pl.ds(i * 176, 176), 0:64] = jnp.where(b == NTB, jnp.zeros_like(yi), yi)


def _t1(features):
    return pl.pallas_call(
        _t1_body,
        grid=(NCAM, NTB + 1),
        in_specs=[pl.BlockSpec((1, C, HB, 176),
                               lambda i, b: (i, 0, jnp.minimum(b, NTB - 1), 0))],
        out_specs=pl.BlockSpec((TBLK, 128), lambda i, b: (i * (NTB + 1) + b, 0)),
        out_shape=jax.ShapeDtypeStruct((NROWS, 128), jnp.float32),
    )(features)


def _t2_body(x_ref, o_ref):
    x = x_ref[0]
    x64 = x[:, 0:64]
    eye = (lax.broadcasted_iota(jnp.int32, (C, C), 0)
           == lax.broadcasted_iota(jnp.int32, (C, C), 1)).astype(jnp.float32)
    for i in range(8):
        xi = x64[i * 200:(i + 1) * 200, :]
        yi = lax.dot_general(eye, xi, (((1,), (1,)), ((), ())),
                             preferred_element_type=jnp.float32,
                             precision=lax.Precision.HIGHEST)
        o_ref[0, :, i, :] = yi


def _t2(outP):
    return pl.pallas_call(
        _t2_body,
        grid=(4, 25),
        in_specs=[pl.BlockSpec((1, 1600, 128), lambda q, b: (q, b, 0))],
        out_specs=pl.BlockSpec((1, C, 8, 200), lambda q, b: (0, q, b, 0)),
        out_shape=jax.ShapeDtypeStruct((1, 256, 200, 200), jnp.float32),
    )(outP)


def _gc_body(g_ref, o_ref):
    cam = lax.broadcasted_iota(jnp.int32, (NCAM, K), 0)
    o_ref[...] = g_ref[...] + cam * CAM_STRIDE


def _gcomb(gather_idx):
    return pl.pallas_call(
        _gc_body,
        out_shape=jax.ShapeDtypeStruct((NCAM, K), jnp.int32),
    )(gather_idx)


def _sc1_body(gidx, sidx, win_hbm,
              src_local, swin, gwin, s_sh, g_sh, sem_i0, sem_i1):
    cid = lax.axis_index("c")
    sid = lax.axis_index("s")
    wid = sid * NCORES + cid
    lo = wid * S_PER_W
    lane = lax.broadcasted_iota(jnp.int32, (16,), 0)

    @pl.when(sid == 0)
    def _stage():
        pltpu.sync_copy(sidx, s_sh)
        pltpu.sync_copy(gidx, g_sh)

    def _init(i, carry):
        r = i * 16 + lane
        pad = (r % NCAM) * CAM_STRIDE + PAD_BASE + (r % TBLK)
        src_local[pl.ds(i * 16, 16)] = pad
        return carry
    lax.fori_loop(0, SRC_PAD // 16, _init, 0)

    plsc.subcore_barrier()

    def _fire(w, half):
        off = half * WIN
        sem = sem_i0 if half == 0 else sem_i1
        pltpu.async_copy(s_sh.at[pl.ds(w * WIN, WIN)],
                         swin.at[pl.ds(off, WIN)], sem)
        pltpu.async_copy(g_sh.at[pl.ds(w * WIN, WIN)],
                         gwin.at[pl.ds(off, WIN)], sem)

    def _drain(half):
        off = half * WIN
        sem = sem_i0 if half == 0 else sem_i1
        pltpu.make_async_copy(s_sh.at[pl.ds(0, WIN)],
                              swin.at[pl.ds(off, WIN)], sem).wait()
        pltpu.make_async_copy(g_sh.at[pl.ds(0, WIN)],
                              gwin.at[pl.ds(off, WIN)], sem).wait()

    def _scan(half):
        base = half * WIN

        def _load(j):
            off0 = base + jnp.minimum(j, VGRP - 1) * 80
            return tuple(swin[pl.ds(off0 + t * 16, 16)] for t in range(5)) \
                + tuple(gwin[pl.ds(off0 + t * 16, 16)] for t in range(5))

        def _vgrp(j, carry):
            nxt = _load(j + 1)
            for t in range(5):
                s_vec, g_vec = carry[t], carry[5 + t]
                p_vec = (s_vec & 3) * (S_TOTAL // 4) + (s_vec >> 2)
                local = p_vec - lo
                m = plsc.bitcast(local, jnp.uint32) < jnp.uint32(S_PER_W)
                plsc.store_scatter(src_local, [local], g_vec, mask=m)
            return nxt
        lax.fori_loop(0, VGRP, _vgrp, _load(0))

    _fire(0, 0)

    def _wpair(u, carry):
        w = u * 2
        _fire(w + 1, 1)
        _drain(0)
        _scan(0)
        _fire(w + 2, 0)
        _drain(1)
        _scan(1)
        return carry
    lax.fori_loop(0, NWIN // 2, _wpair, 0)
    _drain(0)
    _scan(0)

    pltpu.sync_copy(src_local.at[pl.ds(0, S_PER_W)],
                    win_hbm.at[pl.ds(lo, S_PER_W)])


def _sc2_body(win_hbm, featT, out_hbm,
              src_local, rows_a, rows_b, sem_a, sem_b):
    cid = lax.axis_index("c")
    sid = lax.axis_index("s")
    wid = sid * NCORES + cid
    lo = wid * S_PER_W

    pltpu.sync_copy(win_hbm.at[pl.ds(lo, S_PER_W)],
                    src_local.at[pl.ds(0, S_PER_W)])

    def _fire_rows(g, rows, sem):
        for b in range(GCH):
            pltpu.async_copy(
                featT.at[src_local.at[pl.ds(g * GROUP + b * CH, CH)]],
                rows.at[pl.ds(b * CH, CH)], sem)

    def _drain_rows(rows, sem):
        for b in range(GCH):
            pltpu.make_async_copy(
                featT.at[src_local.at[pl.ds(b * CH, CH)]],
                rows.at[pl.ds(b * CH, CH)], sem).wait()

    def _wb(g, rows):
        pltpu.sync_copy(rows, out_hbm.at[pl.ds(lo + g * GROUP, GROUP)])

    _fire_rows(0, rows_a, sem_a)

    def _gpair(t, carry):
        ga = t * 2
        _fire_rows(ga + 1, rows_b, sem_b)
        _drain_rows(rows_a, sem_a)
        _wb(ga, rows_a)
        _fire_rows(ga + 2, rows_a, sem_a)
        _drain_rows(rows_b, sem_b)
        _wb(ga + 1, rows_b)
        return carry
    lax.fori_loop(0, NGRP // 2, _gpair, 0)
    _drain_rows(rows_a, sem_a)
    _wb(NGRP - 1, rows_a)


_SC_MESH = dict(core_axis_name="c", subcore_axis_name="s",
                num_cores=NCORES, num_subcores=NSUB)
_SC_PARAMS = dict(use_tc_tiling_on_sc=False, needs_layout_passes=False)


@functools.cache
def _sc1_call():
    return pl.kernel(
        _sc1_body,
        out_type=jax.ShapeDtypeStruct((S_TOTAL,), jnp.int32),
        mesh=plsc.VectorSubcoreMesh(**_SC_MESH),
        compiler_params=pltpu.CompilerParams(**_SC_PARAMS),
        scratch_types=[
            pltpu.VMEM((SRC_PAD,), jnp.int32),
            pltpu.VMEM((2 * WIN,), jnp.int32),
            pltpu.VMEM((2 * WIN,), jnp.int32),
            pltpu.VMEM_SHARED((NWRITES,), jnp.int32),
            pltpu.VMEM_SHARED((NWRITES,), jnp.int32),
            pltpu.SemaphoreType.DMA,
            pltpu.SemaphoreType.DMA,
        ],
    )


@functools.cache
def _sc2_call():
    return pl.kernel(
        _sc2_body,
        out_type=jax.ShapeDtypeStruct((S_TOTAL, 128), jnp.float32),
        mesh=plsc.VectorSubcoreMesh(**_SC_MESH),
        compiler_params=pltpu.CompilerParams(**_SC_PARAMS),
        scratch_types=[
            pltpu.VMEM((SRC_PAD,), jnp.int32),
            pltpu.VMEM((GROUP, 128), jnp.float32),
            pltpu.VMEM((GROUP, 128), jnp.float32),
            pltpu.SemaphoreType.DMA,
            pltpu.SemaphoreType.DMA,
        ],
    )


def kernel(features, gather_idx, scatter_idx):
    featT = _t1(features)
    gcomb = _gcomb(gather_idx.astype(jnp.int32))
    winners = _sc1_call()(gcomb.reshape(-1),
                          scatter_idx.reshape(-1).astype(jnp.int32))
    outP = _sc2_call()(winners, featT)
    return _t2(outP.reshape(4, S_TOTAL // 4, 128))


# speedup vs baseline: 1.3878x; 1.3878x over previous
import functools

import jax
import jax.numpy as jnp
from jax import lax
from jax.experimental import pallas as pl
from jax.experimental.pallas import tpu as pltpu
from jax.experimental.pallas import tpu_sc as plsc

NCAM = 6
C = 64
HW = 11264
TBLK = 1408
NTB = HW // TBLK
CAM_STRIDE = (NTB + 1) * TBLK
PAD_BASE = HW
NROWS = NCAM * CAM_STRIDE
K = 25000
NWRITES = NCAM * K
S_TOTAL = 160000
NCORES = 2
NSUB = 16
NWORKERS = NCORES * NSUB
S_PER_W = S_TOTAL // NWORKERS
SRC_PAD = 5008
WIN = 2000
NWIN = NWRITES // WIN
VGRP = 25
CH = 40
GCH = 5
GROUP = CH * GCH
NGRP = S_PER_W // GROUP


HB = 8


def _t1_body(f_ref, o_ref):
    b = pl.program_id(1)
    x4 = f_ref[0]
    eye = (lax.broadcasted_iota(jnp.int32, (C, C), 0)
           == lax.broadcasted_iota(jnp.int32, (C, C), 1)).astype(jnp.float32)
    for i in range(HB):
        xi = x4[:, i, :]
        yi = lax.dot_general(xi, eye, (((0,), (0,)), ((), ())),
                             preferred_element_type=jnp.float32,
                             precision=lax.Precision.HIGHEST)
        o_ref[pl.ds(i * 176, 176), :] = jnp.where(b == NTB, jnp.zeros_like(yi), yi)


def _t1(features):
    return pl.pallas_call(
        _t1_body,
        grid=(NCAM, NTB + 1),
        in_specs=[pl.BlockSpec((1, C, HB, 176),
                               lambda i, b: (i, 0, jnp.minimum(b, NTB - 1), 0))],
        out_specs=pl.BlockSpec((TBLK, C), lambda i, b: (i * (NTB + 1) + b, 0)),
        out_shape=jax.ShapeDtypeStruct((NROWS, C), jnp.float32),
    )(features)


def _t2_body(x_ref, o_ref):
    x = x_ref[...]
    eye = (lax.broadcasted_iota(jnp.int32, (256, 256), 0)
           == lax.broadcasted_iota(jnp.int32, (256, 256), 1)).astype(jnp.float32)
    o_ref[...] = lax.dot_general(eye, x, (((0,), (1,)), ((), ())),
                                 preferred_element_type=jnp.float32,
                                 precision=lax.Precision.HIGHEST)


def _t2(outT2):
    rb = 4096
    return pl.pallas_call(
        _t2_body,
        grid=((40000 + rb - 1) // rb,),
        in_specs=[pl.BlockSpec((rb, 256), lambda b: (b, 0))],
        out_specs=pl.BlockSpec((256, rb), lambda b: (0, b)),
        out_shape=jax.ShapeDtypeStruct((256, 40000), jnp.float32),
    )(outT2)


def _gc_body(g_ref, o_ref):
    cam = lax.broadcasted_iota(jnp.int32, (NCAM, K), 0)
    o_ref[...] = g_ref[...] + cam * CAM_STRIDE


def _gcomb(gather_idx):
    return pl.pallas_call(
        _gc_body,
        out_shape=jax.ShapeDtypeStruct((NCAM, K), jnp.int32),
    )(gather_idx)


def _sc1_body(gidx, sidx, win_hbm,
              src_local, swin, gwin, s_sh, g_sh, sem_i0, sem_i1):
    cid = lax.axis_index("c")
    sid = lax.axis_index("s")
    wid = sid * NCORES + cid
    lo = wid * S_PER_W
    lane = lax.broadcasted_iota(jnp.int32, (16,), 0)

    @pl.when(sid == 0)
    def _stage():
        pltpu.sync_copy(sidx, s_sh)
        pltpu.sync_copy(gidx, g_sh)

    def _init(i, carry):
        r = i * 16 + lane
        pad = (r % NCAM) * CAM_STRIDE + PAD_BASE + (r % TBLK)
        src_local[pl.ds(i * 16, 16)] = pad
        return carry
    lax.fori_loop(0, SRC_PAD // 16, _init, 0)

    plsc.subcore_barrier()

    def _fire(w, half):
        off = half * WIN
        sem = sem_i0 if half == 0 else sem_i1
        pltpu.async_copy(s_sh.at[pl.ds(w * WIN, WIN)],
                         swin.at[pl.ds(off, WIN)], sem)
        pltpu.async_copy(g_sh.at[pl.ds(w * WIN, WIN)],
                         gwin.at[pl.ds(off, WIN)], sem)

    def _drain(half):
        off = half * WIN
        sem = sem_i0 if half == 0 else sem_i1
        pltpu.make_async_copy(s_sh.at[pl.ds(0, WIN)],
                              swin.at[pl.ds(off, WIN)], sem).wait()
        pltpu.make_async_copy(g_sh.at[pl.ds(0, WIN)],
                              gwin.at[pl.ds(off, WIN)], sem).wait()

    def _scan(half):
        base = half * WIN

        def _load(j):
            off0 = base + jnp.minimum(j, VGRP - 1) * 80
            return tuple(swin[pl.ds(off0 + t * 16, 16)] for t in range(5)) \
                + tuple(gwin[pl.ds(off0 + t * 16, 16)] for t in range(5))

        def _vgrp(j, carry):
            nxt = _load(j + 1)
            for t in range(5):
                s_vec, g_vec = carry[t], carry[5 + t]
                local = s_vec - lo
                m = plsc.bitcast(local, jnp.uint32) < jnp.uint32(S_PER_W)
                plsc.store_scatter(src_local, [local], g_vec, mask=m)
            return nxt
        lax.fori_loop(0, VGRP, _vgrp, _load(0))

    _fire(0, 0)

    def _wpair(u, carry):
        w = u * 2
        _fire(w + 1, 1)
        _drain(0)
        _scan(0)
        _fire(w + 2, 0)
        _drain(1)
        _scan(1)
        return carry
    lax.fori_loop(0, NWIN // 2, _wpair, 0)
    _drain(0)
    _scan(0)

    pltpu.sync_copy(src_local.at[pl.ds(0, S_PER_W)],
                    win_hbm.at[pl.ds(lo, S_PER_W)])


def _sc2_body(win_hbm, featT, out_hbm,
              src_local, rows_a, rows_b, sem_a, sem_b):
    cid = lax.axis_index("c")
    sid = lax.axis_index("s")
    wid = sid * NCORES + cid
    lo = wid * S_PER_W

    pltpu.sync_copy(win_hbm.at[pl.ds(lo, S_PER_W)],
                    src_local.at[pl.ds(0, S_PER_W)])

    def _fire_rows(g, rows, sem):
        for b in range(GCH):
            pltpu.async_copy(
                featT.at[src_local.at[pl.ds(g * GROUP + b * CH, CH)]],
                rows.at[pl.ds(b * CH, CH)], sem)

    def _drain_rows(rows, sem):
        for b in range(GCH):
            pltpu.make_async_copy(
                featT.at[src_local.at[pl.ds(b * CH, CH)]],
                rows.at[pl.ds(b * CH, CH)], sem).wait()

    def _wb(g, rows):
        pltpu.sync_copy(rows, out_hbm.at[pl.ds(lo + g * GROUP, GROUP)])

    _fire_rows(0, rows_a, sem_a)

    def _gpair(t, carry):
        ga = t * 2
        _fire_rows(ga + 1, rows_b, sem_b)
        _drain_rows(rows_a, sem_a)
        _wb(ga, rows_a)
        _fire_rows(ga + 2, rows_a, sem_a)
        _drain_rows(rows_b, sem_b)
        _wb(ga + 1, rows_b)
        return carry
    lax.fori_loop(0, NGRP // 2, _gpair, 0)
    _drain_rows(rows_a, sem_a)
    _wb(NGRP - 1, rows_a)


_SC_MESH = dict(core_axis_name="c", subcore_axis_name="s",
                num_cores=NCORES, num_subcores=NSUB)
_SC_PARAMS = dict(use_tc_tiling_on_sc=False, needs_layout_passes=False)


@functools.cache
def _sc1_call():
    return pl.kernel(
        _sc1_body,
        out_type=jax.ShapeDtypeStruct((S_TOTAL,), jnp.int32),
        mesh=plsc.VectorSubcoreMesh(**_SC_MESH),
        compiler_params=pltpu.CompilerParams(**_SC_PARAMS),
        scratch_types=[
            pltpu.VMEM((SRC_PAD,), jnp.int32),
            pltpu.VMEM((2 * WIN,), jnp.int32),
            pltpu.VMEM((2 * WIN,), jnp.int32),
            pltpu.VMEM_SHARED((NWRITES,), jnp.int32),
            pltpu.VMEM_SHARED((NWRITES,), jnp.int32),
            pltpu.SemaphoreType.DMA,
            pltpu.SemaphoreType.DMA,
        ],
    )


@functools.cache
def _sc2_call():
    return pl.kernel(
        _sc2_body,
        out_type=jax.ShapeDtypeStruct((S_TOTAL, C), jnp.float32),
        mesh=plsc.VectorSubcoreMesh(**_SC_MESH),
        compiler_params=pltpu.CompilerParams(**_SC_PARAMS),
        scratch_types=[
            pltpu.VMEM((SRC_PAD,), jnp.int32),
            pltpu.VMEM((GROUP, C), jnp.float32),
            pltpu.VMEM((GROUP, C), jnp.float32),
            pltpu.SemaphoreType.DMA,
            pltpu.SemaphoreType.DMA,
        ],
    )


def kernel(features, gather_idx, scatter_idx):
    featT = _t1(features)
    gcomb = _gcomb(gather_idx.astype(jnp.int32))
    winners = _sc1_call()(gcomb.reshape(-1),
                          scatter_idx.reshape(-1).astype(jnp.int32))
    outT = _sc2_call()(winners, featT)
    out2d = _t2(outT.reshape(40000, 256))
    return out2d.reshape(1, 256, 200, 200)
